# baseline (speedup 1.0000x reference)
import functools

import jax
import jax.numpy as jnp
from jax import lax
from jax.experimental import pallas as pl
from jax.experimental.pallas import tpu as pltpu
from jax.experimental.pallas import tpu_sc as plsc

N_EXP = 8
N_EMBD = 1024
NUM_TOKENS = 4096
CAPACITY = 512
ROW = N_EXP * CAPACITY
TB = 512
DROP = 1 << 30

NC, NS = 2, 16
NW = NC * NS
TOK_PER_W = NUM_TOKENS // NW
CHUNK = 16


def _router_body(x_ref, w_ref, uc_ref, t_ref, mask_ref, counts_ref):
    i = pl.program_id(0)

    @pl.when(i == 0)
    def _init():
        counts_ref[...] = jnp.zeros((1, N_EXP), jnp.int32)

    xb = x_ref[...]
    w = w_ref[...]
    logits = jax.lax.dot_general(
        xb, w, (((1,), (1,)), ((), ())),
        preferred_element_type=jnp.float32)

    e_idx = jax.lax.broadcasted_iota(jnp.int32, (TB, N_EXP), 1)
    row_max = jnp.max(logits, axis=1, keepdims=True)
    is_max = logits == row_max
    experts = jnp.min(jnp.where(is_max, e_idx, N_EXP), axis=1,
                      keepdims=True)
    oh = (e_idx == experts).astype(jnp.float32)
    r_i = jax.lax.broadcasted_iota(jnp.int32, (TB, TB), 0)
    c_i = jax.lax.broadcasted_iota(jnp.int32, (TB, TB), 1)
    tri = (r_i > c_i).astype(jnp.float32)
    excl_f = jax.lax.dot_general(
        tri, oh, (((1,), (0,)), ((), ())),
        preferred_element_type=jnp.float32)
    excl = excl_f.astype(jnp.int32)
    base = counts_ref[...]
    ohi = oh.astype(jnp.int32)
    slots = jnp.sum(jnp.where(ohi == 1, excl + base, 0), axis=1,
                    keepdims=True)
    counts_ref[...] = base + jnp.sum(ohi, axis=0, keepdims=True)

    valid = slots < CAPACITY
    t_ref[...] = jnp.where(valid, experts * CAPACITY + slots, DROP)

    s_iota = jax.lax.broadcasted_iota(jnp.int32, (TB, CAPACITY), 1)
    hit_c = (s_iota == slots) & valid
    for e in range(N_EXP):
        mask_ref[:, e, :] = hit_c & (experts == e)

    @pl.when(i == pl.num_programs(0) - 1)
    def _fin():
        uc_ref[...] = jnp.minimum(counts_ref[...], CAPACITY)


def _run_router(x, w_g, interpret=False):
    return pl.pallas_call(
        _router_body,
        grid=(NUM_TOKENS // TB,),
        in_specs=[
            pl.BlockSpec((TB, N_EMBD), lambda i: (i, 0)),
            pl.BlockSpec((N_EXP, N_EMBD), lambda i: (0, 0)),
        ],
        out_specs=[
            pl.BlockSpec((1, N_EXP), lambda i: (0, 0)),
            pl.BlockSpec((TB, 1), lambda i: (i, 0)),
            pl.BlockSpec((TB, N_EXP, CAPACITY), lambda i: (i, 0, 0)),
        ],
        out_shape=[
            jax.ShapeDtypeStruct((1, N_EXP), jnp.int32),
            jax.ShapeDtypeStruct((NUM_TOKENS, 1), jnp.int32),
            jax.ShapeDtypeStruct((NUM_TOKENS, N_EXP, CAPACITY), jnp.bool_),
        ],
        scratch_shapes=[pltpu.VMEM((1, N_EXP), jnp.int32)],
        compiler_params=pltpu.CompilerParams(
            dimension_semantics=("arbitrary",)),
        interpret=interpret,
    )(x, w_g)


def _cb_body(target_hbm, out_hbm, tv, buf):
    wid = lax.axis_index("s") * NC + lax.axis_index("c")
    base_tok = wid * TOK_PER_W
    pltpu.sync_copy(target_hbm.at[pl.ds(base_tok, TOK_PER_W)], tv)

    zeros16 = jnp.zeros((16,), jnp.float32)

    def zbody(j, carry):
        t = jax.lax.shift_right_logical(j, 5)
        base = (j & 31) * 128
        for k in range(8):
            off = base + k * 16
            e = jax.lax.shift_right_logical(off, 9)
            s = off & (CAPACITY - 1)
            buf[t, e, pl.ds(s, 16)] = zeros16
        return carry

    lax.fori_loop(0, CHUNK * ROW // 128, zbody, 0)

    lane = lax.iota(jnp.int32, 16)
    ones = jnp.ones((16,), jnp.float32)
    for c in range(TOK_PER_W // CHUNK):
        t = tv[pl.ds(c * CHUNK, 16)]
        valid = t < ROW
        t = jnp.where(valid, t, 0)
        e = jax.lax.shift_right_logical(t, 9)
        s = t & (CAPACITY - 1)
        plsc.store_scatter(buf, [lane, e, s], ones, mask=valid)
        pltpu.sync_copy(
            buf, out_hbm.at[pl.ds(base_tok + c * CHUNK, CHUNK)])
        plsc.store_scatter(buf, [lane, e, s], zeros16, mask=valid)


@functools.cache
def _cb_kernel():
    return pl.kernel(
        _cb_body,
        out_type=jax.ShapeDtypeStruct((NUM_TOKENS, N_EXP, CAPACITY),
                                      jnp.float32),
        mesh=plsc.VectorSubcoreMesh(core_axis_name="c", subcore_axis_name="s",
                                    num_cores=NC, num_subcores=NS),
        scratch_types=[
            pltpu.VMEM((TOK_PER_W,), jnp.int32),
            pltpu.VMEM((CHUNK, N_EXP, CAPACITY), jnp.float32),
        ],
        compiler_params=pltpu.CompilerParams(needs_layout_passes=False,
                                             use_tc_tiling_on_sc=True),
    )


def kernel(x, w_g):
    uc, target, mask = _run_router(x, w_g)
    cb = _cb_kernel()(target.reshape(NUM_TOKENS))
    return (uc.reshape(N_EXP), cb, mask)


# speedup vs baseline: 1.0018x; 1.0018x over previous
import functools

import jax
import jax.numpy as jnp
from jax import lax
from jax.experimental import pallas as pl
from jax.experimental.pallas import tpu as pltpu
from jax.experimental.pallas import tpu_sc as plsc

N_EXP = 8
N_EMBD = 1024
NUM_TOKENS = 4096
CAPACITY = 512
ROW = N_EXP * CAPACITY
TB = 512
DROP = 1 << 30

NC, NS = 2, 16
NW = NC * NS
TOK_PER_W = NUM_TOKENS // NW
CHUNK = 16


def _router_body(x_ref, w_ref, uc_ref, t_ref, mask_ref, counts_ref):
    i = pl.program_id(0)

    @pl.when(i == 0)
    def _init():
        counts_ref[...] = jnp.zeros((1, N_EXP), jnp.int32)

    xb = x_ref[...]
    w = w_ref[...]
    logits = jax.lax.dot_general(
        xb, w, (((1,), (1,)), ((), ())),
        preferred_element_type=jnp.float32)

    e_idx = jax.lax.broadcasted_iota(jnp.int32, (TB, N_EXP), 1)
    row_max = jnp.max(logits, axis=1, keepdims=True)
    is_max = logits == row_max
    experts = jnp.min(jnp.where(is_max, e_idx, N_EXP), axis=1,
                      keepdims=True)
    oh = (e_idx == experts).astype(jnp.float32)
    r_i = jax.lax.broadcasted_iota(jnp.int32, (TB, TB), 0)
    c_i = jax.lax.broadcasted_iota(jnp.int32, (TB, TB), 1)
    tri = (r_i > c_i).astype(jnp.float32)
    excl_f = jax.lax.dot_general(
        tri, oh, (((1,), (0,)), ((), ())),
        preferred_element_type=jnp.float32)
    excl = excl_f.astype(jnp.int32)
    base = counts_ref[...]
    ohi = oh.astype(jnp.int32)
    slots = jnp.sum(jnp.where(ohi == 1, excl + base, 0), axis=1,
                    keepdims=True)
    counts_ref[...] = base + jnp.sum(ohi, axis=0, keepdims=True)

    valid = slots < CAPACITY
    t_ref[...] = jnp.where(valid, experts * CAPACITY + slots, DROP)

    s_iota = jax.lax.broadcasted_iota(jnp.int32, (TB, CAPACITY), 1)
    hit_c = (s_iota == slots) & valid
    for e in range(N_EXP):
        mask_ref[:, e, :] = hit_c & (experts == e)

    @pl.when(i == pl.num_programs(0) - 1)
    def _fin():
        uc_ref[...] = jnp.minimum(counts_ref[...], CAPACITY)


def _run_router(x, w_g, interpret=False):
    return pl.pallas_call(
        _router_body,
        grid=(NUM_TOKENS // TB,),
        in_specs=[
            pl.BlockSpec((TB, N_EMBD), lambda i: (i, 0)),
            pl.BlockSpec((N_EXP, N_EMBD), lambda i: (0, 0)),
        ],
        out_specs=[
            pl.BlockSpec((1, N_EXP), lambda i: (0, 0)),
            pl.BlockSpec((TB, 1), lambda i: (i, 0)),
            pl.BlockSpec((TB, N_EXP, CAPACITY), lambda i: (i, 0, 0)),
        ],
        out_shape=[
            jax.ShapeDtypeStruct((1, N_EXP), jnp.int32),
            jax.ShapeDtypeStruct((NUM_TOKENS, 1), jnp.int32),
            jax.ShapeDtypeStruct((NUM_TOKENS, N_EXP, CAPACITY), jnp.bool_),
        ],
        scratch_shapes=[pltpu.VMEM((1, N_EXP), jnp.int32)],
        compiler_params=pltpu.CompilerParams(
            dimension_semantics=("arbitrary",),
            skip_device_barrier=True),
        interpret=interpret,
    )(x, w_g)


def _cb_body(target_hbm, out_hbm, tv, buf):
    wid = lax.axis_index("s") * NC + lax.axis_index("c")
    base_tok = wid * TOK_PER_W
    pltpu.sync_copy(target_hbm.at[pl.ds(base_tok, TOK_PER_W)], tv)

    zeros16 = jnp.zeros((16,), jnp.float32)

    def zbody(j, carry):
        t = jax.lax.shift_right_logical(j, 5)
        base = (j & 31) * 128
        for k in range(8):
            off = base + k * 16
            e = jax.lax.shift_right_logical(off, 9)
            s = off & (CAPACITY - 1)
            buf[t, e, pl.ds(s, 16)] = zeros16
        return carry

    lax.fori_loop(0, CHUNK * ROW // 128, zbody, 0)

    lane = lax.iota(jnp.int32, 16)
    ones = jnp.ones((16,), jnp.float32)
    for c in range(TOK_PER_W // CHUNK):
        t = tv[pl.ds(c * CHUNK, 16)]
        valid = t < ROW
        t = jnp.where(valid, t, 0)
        e = jax.lax.shift_right_logical(t, 9)
        s = t & (CAPACITY - 1)
        plsc.store_scatter(buf, [lane, e, s], ones, mask=valid)
        pltpu.sync_copy(
            buf, out_hbm.at[pl.ds(base_tok + c * CHUNK, CHUNK)])
        plsc.store_scatter(buf, [lane, e, s], zeros16, mask=valid)


@functools.cache
def _cb_kernel():
    return pl.kernel(
        _cb_body,
        out_type=jax.ShapeDtypeStruct((NUM_TOKENS, N_EXP, CAPACITY),
                                      jnp.float32),
        mesh=plsc.VectorSubcoreMesh(core_axis_name="c", subcore_axis_name="s",
                                    num_cores=NC, num_subcores=NS),
        scratch_types=[
            pltpu.VMEM((TOK_PER_W,), jnp.int32),
            pltpu.VMEM((CHUNK, N_EXP, CAPACITY), jnp.float32),
        ],
        compiler_params=pltpu.CompilerParams(needs_layout_passes=False,
                                             use_tc_tiling_on_sc=True,
                                             skip_device_barrier=True),
    )


def kernel(x, w_g):
    uc, target, mask = _run_router(x, w_g)
    cb = _cb_kernel()(target.reshape(NUM_TOKENS))
    return (uc.reshape(N_EXP), cb, mask)


# speedup vs baseline: 1.0195x; 1.0176x over previous
import functools

import jax
import jax.numpy as jnp
from jax import lax
from jax.experimental import pallas as pl
from jax.experimental.pallas import tpu as pltpu
from jax.experimental.pallas import tpu_sc as plsc

N_EXP = 8
N_EMBD = 1024
NUM_TOKENS = 4096
CAPACITY = 512
ROW = N_EXP * CAPACITY
TB = 512
DROP = 1 << 30

NC, NS = 2, 16
NW = NC * NS
TOK_PER_W = NUM_TOKENS // NW
CHUNK = 16


def _router_body(x_ref, w_ref, uc_ref, t_ref, mask_ref, counts_ref):
    i = pl.program_id(0)

    @pl.when(i == 0)
    def _init():
        counts_ref[...] = jnp.zeros((1, N_EXP), jnp.int32)

    xb = x_ref[...]
    w = w_ref[...]
    logits = jax.lax.dot_general(
        xb, w, (((1,), (1,)), ((), ())),
        preferred_element_type=jnp.float32)

    e_idx = jax.lax.broadcasted_iota(jnp.int32, (TB, N_EXP), 1)
    row_max = jnp.max(logits, axis=1, keepdims=True)
    is_max = logits == row_max
    experts = jnp.min(jnp.where(is_max, e_idx, N_EXP), axis=1,
                      keepdims=True)
    oh = (e_idx == experts).astype(jnp.float32)
    r_i = jax.lax.broadcasted_iota(jnp.int32, (TB, TB), 0)
    c_i = jax.lax.broadcasted_iota(jnp.int32, (TB, TB), 1)
    tri = (r_i > c_i).astype(jnp.float32)
    excl_f = jax.lax.dot_general(
        tri, oh, (((1,), (0,)), ((), ())),
        preferred_element_type=jnp.float32)
    excl = excl_f.astype(jnp.int32)
    base = counts_ref[...]
    ohi = oh.astype(jnp.int32)
    slots = jnp.sum(jnp.where(ohi == 1, excl + base, 0), axis=1,
                    keepdims=True)
    counts_ref[...] = base + jnp.sum(ohi, axis=0, keepdims=True)

    valid = slots < CAPACITY
    t_ref[...] = jnp.where(valid, experts * CAPACITY + slots, DROP)

    s_iota = jax.lax.broadcasted_iota(jnp.int32, (TB, CAPACITY), 1)
    hit_c = (s_iota == slots) & valid
    for e in range(N_EXP):
        mask_ref[:, e, :] = hit_c & (experts == e)

    @pl.when(i == pl.num_programs(0) - 1)
    def _fin():
        uc_ref[...] = jnp.minimum(counts_ref[...], CAPACITY)


def _run_router(x, w_g, interpret=False):
    return pl.pallas_call(
        _router_body,
        grid=(NUM_TOKENS // TB,),
        in_specs=[
            pl.BlockSpec((TB, N_EMBD), lambda i: (i, 0)),
            pl.BlockSpec((N_EXP, N_EMBD), lambda i: (0, 0)),
        ],
        out_specs=[
            pl.BlockSpec((1, N_EXP), lambda i: (0, 0)),
            pl.BlockSpec((TB, 1), lambda i: (i, 0)),
            pl.BlockSpec((TB, N_EXP, CAPACITY), lambda i: (i, 0, 0)),
        ],
        out_shape=[
            jax.ShapeDtypeStruct((1, N_EXP), jnp.int32),
            jax.ShapeDtypeStruct((NUM_TOKENS, 1), jnp.int32),
            jax.ShapeDtypeStruct((NUM_TOKENS, N_EXP, CAPACITY), jnp.bool_),
        ],
        scratch_shapes=[pltpu.VMEM((1, N_EXP), jnp.int32)],
        compiler_params=pltpu.CompilerParams(
            dimension_semantics=("arbitrary",)),
        interpret=interpret,
    )(x, w_g)


def _cb_body(target_hbm, out_hbm, tv, buf):
    wid = lax.axis_index("s") * NC + lax.axis_index("c")
    base_tok = wid * TOK_PER_W
    pltpu.sync_copy(target_hbm.at[pl.ds(base_tok, TOK_PER_W)], tv)

    zeros16 = jnp.zeros((16,), jnp.float32)

    def zbody(j, carry):
        t = jax.lax.shift_right_logical(j, 8)
        e = jax.lax.shift_right_logical(j, 5) & 7
        s = (j & 31) * 16
        buf[t, e, pl.ds(s, 16)] = zeros16
        return carry

    lax.fori_loop(0, CHUNK * ROW // 16, zbody, 0)

    lane = lax.iota(jnp.int32, 16)
    ones = jnp.ones((16,), jnp.float32)
    for c in range(TOK_PER_W // CHUNK):
        t = tv[pl.ds(c * CHUNK, 16)]
        valid = t < ROW
        t = jnp.where(valid, t, 0)
        e = jax.lax.shift_right_logical(t, 9)
        s = t & (CAPACITY - 1)
        plsc.store_scatter(buf, [lane, e, s], ones, mask=valid)
        pltpu.sync_copy(
            buf, out_hbm.at[pl.ds(base_tok + c * CHUNK, CHUNK)])
        plsc.store_scatter(buf, [lane, e, s], zeros16, mask=valid)


@functools.cache
def _cb_kernel():
    return pl.kernel(
        _cb_body,
        out_type=jax.ShapeDtypeStruct((NUM_TOKENS, N_EXP, CAPACITY),
                                      jnp.float32),
        mesh=plsc.VectorSubcoreMesh(core_axis_name="c", subcore_axis_name="s",
                                    num_cores=NC, num_subcores=NS),
        scratch_types=[
            pltpu.VMEM((TOK_PER_W,), jnp.int32),
            pltpu.VMEM((CHUNK, N_EXP, CAPACITY), jnp.float32),
        ],
        compiler_params=pltpu.CompilerParams(needs_layout_passes=False),
    )


def kernel(x, w_g):
    uc, target, mask = _run_router(x, w_g)
    cb = _cb_kernel()(target.reshape(NUM_TOKENS))
    return (uc.reshape(N_EXP), cb, mask)
